# Initial kernel scaffold; baseline (speedup 1.0000x reference)
import jax
import jax.numpy as jnp
from jax.experimental import pallas as pl


def kernel(corr, mc, dt):
    raise NotImplementedError("write your pallas kernel here")


# speedup vs baseline: 5.0241x; 5.0241x over previous
import jax
import jax.numpy as jnp
from jax import lax
from jax.experimental import pallas as pl
from jax.experimental.pallas import tpu as pltpu

_BINS = 100
_HMIN = 60.0
_HMAX = 120.0

_C = 512
_BR = 256
_CHUNK = _BR * _C


def _bin_index(x):
    t = (x - _HMIN) * _BINS / (_HMAX - _HMIN)
    i0 = jnp.clip(jnp.floor(t).astype(jnp.int32), 0, _BINS - 1)
    valid = (x >= _HMIN) & (x <= _HMAX)
    return jnp.where(valid, i0, _BINS)


def _body(cols_ref, out_ref, acc_mc, acc_dt):
    i = pl.program_id(0)
    n = pl.num_programs(0)

    @pl.when(i == 0)
    def _init():
        acc_mc[...] = jnp.zeros_like(acc_mc)
        acc_dt[...] = jnp.zeros_like(acc_dt)

    c0 = cols_ref[0]
    c1 = cols_ref[1]
    m0 = cols_ref[2]
    m1 = cols_ref[3]
    f1 = cols_ref[4]
    f2 = cols_ref[5]
    e1 = cols_ref[6]
    e2 = cols_ref[7]
    x_dt = cols_ref[8]

    pt1 = c0 * m0
    pt2 = c1 * m1
    de = e1 - e2
    cosh_de = 0.5 * (jnp.exp(de) + jnp.exp(-de))
    mz2 = 2.0 * pt1 * pt2 * (cosh_de - jnp.cos(f1 - f2))
    mz = jnp.sqrt(jnp.maximum(mz2, 0.0))

    idx_mc = _bin_index(mz)
    idx_dt = _bin_index(x_dt)

    def bin_step(b, carry):
        pm = jnp.sum((idx_mc == b).astype(jnp.float32), axis=0, keepdims=True)
        pd = jnp.sum((idx_dt == b).astype(jnp.float32), axis=0, keepdims=True)
        acc_mc[pl.ds(b, 1)] += pm.reshape(1, 1, _C)
        acc_dt[pl.ds(b, 1)] += pd.reshape(1, 1, _C)
        return carry

    lax.fori_loop(0, _BINS + 1, bin_step, 0)

    @pl.when(i == n - 1)
    def _finish():
        hm = jnp.sum(acc_mc[...], axis=(1, 2), keepdims=False).reshape(104, 1)
        hd = jnp.sum(acc_dt[...], axis=(1, 2), keepdims=False).reshape(104, 1)
        rid = lax.broadcasted_iota(jnp.int32, (104, 1), 0)
        t = jnp.where(rid < _BINS, hd, 0.0)
        pw = jnp.where(t > 0.0, t * (jnp.log(jnp.where(t > 0.0, t, 1.0)) - hm), 0.0)
        out_ref[...] = (jnp.sum(pw) / float(_BINS)).reshape(1, 1)


def kernel(corr, mc, dt):
    n = corr.shape[0]
    nblk = -(-n // _CHUNK)
    npad = nblk * _CHUNK
    pad = npad - n

    def prep(col):
        return jnp.pad(col, (0, pad))

    cols = jnp.stack([
        prep(corr[:, 0]),
        prep(corr[:, 1]),
        prep(mc[:, 0]),
        prep(mc[:, 1]),
        prep(mc[:, 4]),
        prep(mc[:, 5]),
        prep(mc[:, 6]),
        prep(mc[:, 7]),
        prep(dt[:, 8]),
    ]).reshape(9, nblk * _BR, _C)

    out = pl.pallas_call(
        _body,
        grid=(nblk,),
        in_specs=[pl.BlockSpec((9, _BR, _C), lambda i: (0, i, 0))],
        out_specs=pl.BlockSpec((1, 1), lambda i: (0, 0)),
        out_shape=jax.ShapeDtypeStruct((1, 1), jnp.float32),
        scratch_shapes=[
            pltpu.VMEM((104, 1, _C), jnp.float32),
            pltpu.VMEM((104, 1, _C), jnp.float32),
        ],
    )(cols)
    return out[0, 0]


# speedup vs baseline: 6.2602x; 1.2460x over previous
import functools

import jax
import jax.numpy as jnp
from jax import lax
from jax.experimental import pallas as pl
from jax.experimental.pallas import tpu as pltpu
from jax.experimental.pallas import tpu_sc as plsc

_BINS = 100
_HMIN = 60.0
_HMAX = 120.0

_C = 512
_BR = 256
_CHUNK = _BR * _C

_NC = 2
_NS = 16
_NW = _NC * _NS
_L = 16

_DT_OFF = 128


def _bin_index(x):
    t = (x - _HMIN) * _BINS / (_HMAX - _HMIN)
    i0 = jnp.clip(jnp.floor(t).astype(jnp.int32), 0, _BINS - 1)
    valid = (x >= _HMIN) & (x <= _HMAX)
    return jnp.where(valid, i0, _BINS)


def _idx_body(cols_ref, imc_ref, idt_ref):
    c0 = cols_ref[0]
    c1 = cols_ref[1]
    m0 = cols_ref[2]
    m1 = cols_ref[3]
    f1 = cols_ref[4]
    f2 = cols_ref[5]
    e1 = cols_ref[6]
    e2 = cols_ref[7]
    x_dt = cols_ref[8]

    pt1 = c0 * m0
    pt2 = c1 * m1
    de = e1 - e2
    cosh_de = 0.5 * (jnp.exp(de) + jnp.exp(-de))
    mz2 = 2.0 * pt1 * pt2 * (cosh_de - jnp.cos(f1 - f2))
    mz = jnp.sqrt(jnp.maximum(mz2, 0.0))

    imc_ref[...] = _bin_index(mz)
    idt_ref[...] = _bin_index(x_dt)


def _sc_hist_body(imc_hbm, idt_hbm, out_hbm, buf, hist):
    wid = lax.axis_index("s") * _NC + lax.axis_index("c")
    nw = buf.shape[0]
    unroll = 16
    span = unroll * _L

    lane = lax.broadcasted_iota(jnp.int32, (_L,), 0)
    ones = jnp.full((_L,), 1.0, jnp.float32)
    zeros = jnp.zeros((_L,), jnp.float32)

    for k in range(_L * 256 // _L):
        hist[pl.ds(k * _L, _L)] = zeros

    def scan_slice(src_hbm, lane_base):
        pltpu.sync_copy(src_hbm.at[pl.ds(wid * nw, nw)], buf)

        def chunk_body(j, carry):
            for u in range(unroll):
                v = buf[pl.ds(j * span + u * _L, _L)]
                plsc.addupdate_scatter(hist, [lane_base + v], ones)
            return carry

        lax.fori_loop(0, nw // span, chunk_body, 0)

    scan_slice(imc_hbm, lane * 256)
    scan_slice(idt_hbm, lane * 256 + _DT_OFF)

    pltpu.sync_copy(hist, out_hbm.at[wid])


def _kl_body(h_ref, out_ref):
    s = jnp.sum(h_ref[...], axis=0, keepdims=True)
    hm = s[:, 0:_BINS]
    hd = s[:, _DT_OFF:_DT_OFF + _BINS]
    pw = jnp.where(hd > 0.0, hd * (jnp.log(jnp.where(hd > 0.0, hd, 1.0)) - hm), 0.0)
    out_ref[...] = (jnp.sum(pw) / float(_BINS)).reshape(1, 1)


def kernel(corr, mc, dt):
    n = corr.shape[0]
    nblk = -(-n // _CHUNK)
    npad = nblk * _CHUNK
    pad = npad - n
    rows = nblk * _BR

    def prep(col):
        return jnp.pad(col, (0, pad))

    cols = jnp.stack([
        prep(corr[:, 0]),
        prep(corr[:, 1]),
        prep(mc[:, 0]),
        prep(mc[:, 1]),
        prep(mc[:, 4]),
        prep(mc[:, 5]),
        prep(mc[:, 6]),
        prep(mc[:, 7]),
        prep(dt[:, 8]),
    ]).reshape(9, rows, _C)

    imc, idt = pl.pallas_call(
        _idx_body,
        grid=(nblk,),
        in_specs=[pl.BlockSpec((9, _BR, _C), lambda i: (0, i, 0))],
        out_specs=[
            pl.BlockSpec((_BR, _C), lambda i: (i, 0)),
            pl.BlockSpec((_BR, _C), lambda i: (i, 0)),
        ],
        out_shape=[
            jax.ShapeDtypeStruct((rows, _C), jnp.int32),
            jax.ShapeDtypeStruct((rows, _C), jnp.int32),
        ],
    )(cols)

    nwords = npad // _NW
    sc_hist = functools.partial(
        pl.kernel,
        mesh=plsc.VectorSubcoreMesh(core_axis_name="c", subcore_axis_name="s"),
        out_type=jax.ShapeDtypeStruct((_NW, _L * 256), jnp.float32),
        scratch_types=[
            pltpu.VMEM((nwords,), jnp.int32),
            pltpu.VMEM((_L * 256,), jnp.float32),
        ],
        compiler_params=pltpu.CompilerParams(needs_layout_passes=False),
    )(_sc_hist_body)
    hists = sc_hist(imc.reshape(-1), idt.reshape(-1))

    out = pl.pallas_call(
        _kl_body,
        in_specs=[pl.BlockSpec((_NW * _L, 256), lambda: (0, 0))],
        out_specs=pl.BlockSpec((1, 1), lambda: (0, 0)),
        out_shape=jax.ShapeDtypeStruct((1, 1), jnp.float32),
    )(hists.reshape(_NW * _L, 256))
    return out[0, 0]
